# baseline (speedup 1.0000x reference)
import functools

import jax
import jax.numpy as jnp
from jax import lax
from jax.experimental import pallas as pl
from jax.experimental.pallas import tpu as pltpu
from jax.experimental.pallas import tpu_sc as plsc

_N = 10000
_D = 256
_OUT = 64
_E = 160000

_TILES = 16
_NCORES = 2
_CH = 128
_NCH = 80
_EPAD = _TILES * _CH * _NCH
_ROWS = _TILES * _NCH
_NPAD = 10240
_NZ = _NPAD // _TILES

_BLK = 1024

_sc_mesh = plsc.VectorSubcoreMesh(core_axis_name="c", subcore_axis_name="s")


@functools.partial(
    pl.kernel,
    out_type=jax.ShapeDtypeStruct((_NCORES * _NPAD, 16), jnp.float32),
    mesh=_sc_mesh,
    compiler_params=pltpu.CompilerParams(needs_layout_passes=False),
    scratch_types=[
        pltpu.VMEM_SHARED((_TILES, _NPAD), jnp.float32),
        pltpu.VMEM((_NPAD,), jnp.float32),
        pltpu.VMEM((_NCH, _CH), jnp.int32),
        pltpu.VMEM((_NZ,), jnp.float32),
        pltpu.VMEM((_NZ,), jnp.float32),
        pltpu.VMEM((_NZ, 16), jnp.float32),
    ],
)
def _deg_kernel(idx_hbm, zeros_hbm, out_hbm,
                spbuf, hist, idx, accv, tmpv, outrows):
    c = lax.axis_index("c")
    s = lax.axis_index("s")
    pltpu.sync_copy(zeros_hbm, hist)
    pltpu.sync_copy(idx_hbm.at[pl.ds((c * _TILES + s) * _NCH, _NCH)], idx)
    ones = jnp.ones((16,), jnp.float32)
    zeros = jnp.zeros((16,), jnp.float32)

    @pl.loop(0, _NCH)
    def _(j):
        for k in range(_CH // 16):
            iv = idx[j, pl.ds(k * 16, 16)]
            plsc.addupdate_scatter(hist, [iv], ones)

    pltpu.sync_copy(hist, spbuf.at[s])
    plsc.subcore_barrier()

    for m in range(_NZ // 16):
        accv[pl.ds(m * 16, 16)] = zeros
    for t in range(_TILES):
        pltpu.sync_copy(spbuf.at[t, pl.ds(s * _NZ, _NZ)], tmpv)
        for m in range(_NZ // 16):
            sl = pl.ds(m * 16, 16)
            accv[sl] = accv[sl] + tmpv[sl]

    iota = lax.iota(jnp.int32, 16)
    for m in range(_NZ // 16):
        v = accv[pl.ds(m * 16, 16)]
        rows = jnp.full((16,), m * 16, jnp.int32) + iota
        for r in range(16):
            plsc.store_scatter(outrows, [rows, jnp.full((16,), r, jnp.int32)], v)

    pltpu.sync_copy(outrows,
                    out_hbm.at[pl.ds(c * _NPAD + s * _NZ, _NZ)])


_ACC = 10112
_NZA = _ACC // _TILES
_PC = 40


def _make_agg(phases, split_sidx):
    @functools.partial(
        pl.kernel,
        out_type=jax.ShapeDtypeStruct((_NCORES * _NPAD, 128), jnp.float32),
        mesh=_sc_mesh,
        scratch_types=[
            pltpu.VMEM_SHARED((_ACC, 128), jnp.float32),
            pltpu.VMEM((_PC, _CH), jnp.int32),
            pltpu.VMEM((_PC, _CH), jnp.int32),
            pltpu.VMEM((_CH, 128), jnp.float32),
            pltpu.VMEM((_CH, 128), jnp.float32),
            pltpu.SemaphoreType.DMA,
            pltpu.SemaphoreType.DMA,
            pltpu.SemaphoreType.DMA,
            pltpu.SemaphoreType.DMA,
        ],
    )
    def agg(p_hbm, gidx_hbm, sidx_hbm, zeros_hbm, out_hbm,
            acc, gidx, sidx, rows0, rows1, gs0, gs1, ss0, ss1):
        c = lax.axis_index("c")
        s = lax.axis_index("s")
        w = c * _TILES + s
        pltpu.sync_copy(zeros_hbm, acc.at[pl.ds(s * _NZA, _NZA)])
        plsc.subcore_barrier()

        def wait_g(buf, gsem):
            pltpu.make_async_copy(p_hbm.at[gidx.at[0]], buf, gsem).wait()

        def wait_s(buf, ssem):
            pltpu.make_async_copy(buf, acc.at[sidx.at[0]], ssem).wait()

        for p in range(phases):
            pltpu.sync_copy(gidx_hbm.at[pl.ds((w * phases + p) * _PC, _PC)],
                            gidx)
            sb = ((w if split_sidx else s) * phases + p) * _PC
            pltpu.sync_copy(sidx_hbm.at[pl.ds(sb, _PC)], sidx)

            pltpu.async_copy(p_hbm.at[gidx.at[0]], rows0, gs0)
            wait_g(rows0, gs0)
            pltpu.async_copy(rows0, acc.at[sidx.at[0]], ss0, add=True)
            pltpu.async_copy(p_hbm.at[gidx.at[1]], rows1, gs1)
            wait_g(rows1, gs1)
            pltpu.async_copy(rows1, acc.at[sidx.at[1]], ss1, add=True)
            wait_s(rows0, ss0)
            pltpu.async_copy(p_hbm.at[gidx.at[2]], rows0, gs0)

            @pl.loop(1, _PC // 2 - 1)
            def _(jj):
                j0 = jj * 2
                wait_g(rows0, gs0)
                pltpu.async_copy(rows0, acc.at[sidx.at[j0]], ss0, add=True)
                wait_s(rows1, ss1)
                pltpu.async_copy(p_hbm.at[gidx.at[j0 + 1]], rows1, gs1)
                wait_g(rows1, gs1)
                pltpu.async_copy(rows1, acc.at[sidx.at[j0 + 1]], ss1, add=True)
                wait_s(rows0, ss0)
                pltpu.async_copy(p_hbm.at[gidx.at[j0 + 2]], rows0, gs0)

            wait_g(rows0, gs0)
            pltpu.async_copy(rows0, acc.at[sidx.at[_PC - 2]], ss0, add=True)
            wait_s(rows1, ss1)
            pltpu.async_copy(p_hbm.at[gidx.at[_PC - 1]], rows1, gs1)
            wait_g(rows1, gs1)
            pltpu.async_copy(rows1, acc.at[sidx.at[_PC - 1]], ss1, add=True)
            wait_s(rows0, ss0)
            wait_s(rows1, ss1)

        plsc.subcore_barrier()
        pltpu.sync_copy(acc.at[pl.ds(s * _NZA, _NZA)],
                        out_hbm.at[pl.ds(c * _NPAD + s * _NZA, _NZA)])

    return agg


_NCH2 = _NCH // 2
_agg128 = _make_agg(phases=2, split_sidx=False)
_agg2 = _make_agg(phases=1, split_sidx=True)


def _norm_col(deg16):
    n = jnp.where(deg16 > 0.0, lax.rsqrt(jnp.maximum(deg16, 1.0)), 0.0)
    return n[:, :1]


def _tc1_body(x_ref, degs_ref, w1_ref, w2_ref, wc_ref, p1_ref, wc2_ref):
    ns = _norm_col(degs_ref[0])
    p1 = jnp.dot(x_ref[...] * ns, w1_ref[...],
                 preferred_element_type=jnp.float32)
    p1_ref[0] = p1[:, :128]
    p1_ref[1] = p1[:, 128:]
    wc2_ref[...] = jnp.dot(w2_ref[...], wc_ref[...],
                           preferred_element_type=jnp.float32)


_tc1 = pl.pallas_call(
    _tc1_body,
    grid=(_NPAD // _BLK,),
    in_specs=[
        pl.BlockSpec((_BLK, _D), lambda i: (i, 0)),
        pl.BlockSpec((1, _BLK, 16), lambda i: (0, i, 0)),
        pl.BlockSpec((_D, _D), lambda i: (0, 0)),
        pl.BlockSpec((_D, _D), lambda i: (0, 0)),
        pl.BlockSpec((_D, _OUT), lambda i: (0, 0)),
    ],
    out_specs=[
        pl.BlockSpec((2, _BLK, 128), lambda i: (0, i, 0)),
        pl.BlockSpec((_D, _OUT), lambda i: (0, 0)),
    ],
    out_shape=[
        jax.ShapeDtypeStruct((2, _NPAD, 128), jnp.float32),
        jax.ShapeDtypeStruct((_D, _OUT), jnp.float32),
    ],
)


def _tc2_body(a_ref, degs_ref, degd_ref, b1_ref, g1_ref, be1_ref, wc2_ref,
              p2_ref):
    ns = _norm_col(degs_ref[0])
    nd = _norm_col(degd_ref[0])
    h0 = a_ref[0] * nd + b1_ref[0:1, :128]
    h1 = a_ref[1] * nd + b1_ref[0:1, 128:]
    mu = (jnp.sum(h0, -1, keepdims=True) +
          jnp.sum(h1, -1, keepdims=True)) * (1.0 / _D)
    c0 = h0 - mu
    c1 = h1 - mu
    var = (jnp.sum(c0 * c0, -1, keepdims=True) +
           jnp.sum(c1 * c1, -1, keepdims=True)) * (1.0 / _D)
    inv = lax.rsqrt(var + 1e-5)
    y0 = jnp.maximum(c0 * inv * g1_ref[0:1, :128] + be1_ref[0:1, :128],
                     0.0) * ns
    y1 = jnp.maximum(c1 * inv * g1_ref[0:1, 128:] + be1_ref[0:1, 128:],
                     0.0) * ns
    p2 = (jnp.dot(y0, wc2_ref[:128], preferred_element_type=jnp.float32) +
          jnp.dot(y1, wc2_ref[128:], preferred_element_type=jnp.float32))
    p2_ref[...] = jnp.concatenate(
        [p2, jnp.zeros((p2.shape[0], 128 - _OUT), jnp.float32)], axis=-1)


_tc2 = pl.pallas_call(
    _tc2_body,
    grid=(_NPAD // _BLK,),
    in_specs=[
        pl.BlockSpec((2, _BLK, 128), lambda i: (0, i, 0)),
        pl.BlockSpec((1, _BLK, 16), lambda i: (0, i, 0)),
        pl.BlockSpec((1, _BLK, 16), lambda i: (1, i, 0)),
        pl.BlockSpec((1, _D), lambda i: (0, 0)),
        pl.BlockSpec((1, _D), lambda i: (0, 0)),
        pl.BlockSpec((1, _D), lambda i: (0, 0)),
        pl.BlockSpec((_D, _OUT), lambda i: (0, 0)),
    ],
    out_specs=pl.BlockSpec((_BLK, 128), lambda i: (i, 0)),
    out_shape=jax.ShapeDtypeStruct((_NPAD, 128), jnp.float32),
)


def _tc3_body(a_ref, degd_ref, b2_ref, wc_ref, bc_ref, out_ref):
    nd = _norm_col(degd_ref[0])
    a2 = (a_ref[0] + a_ref[1])[:, :_OUT] * nd
    bc2 = jnp.dot(b2_ref[...], wc_ref[...],
                  preferred_element_type=jnp.float32) + bc_ref[...]
    out_ref[...] = a2 + bc2


_tc3 = pl.pallas_call(
    _tc3_body,
    grid=(_NPAD // _BLK,),
    in_specs=[
        pl.BlockSpec((2, _BLK, 128), lambda i: (0, i, 0)),
        pl.BlockSpec((1, _BLK, 16), lambda i: (1, i, 0)),
        pl.BlockSpec((1, _D), lambda i: (0, 0)),
        pl.BlockSpec((_D, _OUT), lambda i: (0, 0)),
        pl.BlockSpec((1, _OUT), lambda i: (0, 0)),
    ],
    out_specs=pl.BlockSpec((_BLK, _OUT), lambda i: (i, 0)),
    out_shape=jax.ShapeDtypeStruct((_NPAD, _OUT), jnp.float32),
)


def kernel(x, edge_index, W1, b1, g1, be1, W2, b2, Wc, bc):
    src = edge_index[0]
    dst = edge_index[1]
    pad = jnp.full((_EPAD - _E,), _N, jnp.int32)
    srcp = jnp.concatenate([src, pad])
    dstp = jnp.concatenate([dst, pad])

    deg_idx = jnp.concatenate([srcp, dstp]).reshape(2 * _ROWS, _CH)
    gidx1 = jnp.concatenate([srcp, srcp + _NPAD]).reshape(2 * _ROWS, _CH)
    src_rows = srcp.reshape(_ROWS, _CH)
    dst_rows = dstp.reshape(_ROWS, _CH)

    zeros1d = jnp.zeros((_NPAD,), jnp.float32)
    zeros128 = jnp.zeros((_NZA, 128), jnp.float32)

    degs = _deg_kernel(deg_idx, zeros1d).reshape(_NCORES, _NPAD, 16)

    xp = jnp.pad(x, ((0, _NPAD - _N), (0, 0)))
    p1, wc2 = _tc1(xp, degs, W1, W2, Wc)

    a1 = _agg128(p1.reshape(2 * _NPAD, 128), gidx1, dst_rows,
                 zeros128).reshape(2, _NPAD, 128)

    p2 = _tc2(a1, degs, degs, b1.reshape(1, _D), g1.reshape(1, _D),
              be1.reshape(1, _D), wc2)

    a2 = _agg2(p2, src_rows, dst_rows, zeros128).reshape(2, _NPAD, 128)

    logits = _tc3(a2, degs, b2.reshape(1, _D), Wc, bc.reshape(1, _OUT))
    return logits[:_N]


# speedup vs baseline: 2.3176x; 2.3176x over previous
import functools

import jax
import jax.numpy as jnp
from jax import lax
from jax.experimental import pallas as pl
from jax.experimental.pallas import tpu as pltpu
from jax.experimental.pallas import tpu_sc as plsc

_N = 10000
_D = 256
_OUT = 64
_E = 160000

_TILES = 16
_NCORES = 2
_CH = 128
_NCH = 80
_EPAD = _TILES * _CH * _NCH
_ROWS = _TILES * _NCH
_NPAD = 10240
_NZ = _NPAD // _TILES

_BLK = 1024

_sc_mesh = plsc.VectorSubcoreMesh(core_axis_name="c", subcore_axis_name="s")


@functools.partial(
    pl.kernel,
    out_type=jax.ShapeDtypeStruct((_NCORES * _NPAD, 16), jnp.float32),
    mesh=_sc_mesh,
    compiler_params=pltpu.CompilerParams(needs_layout_passes=False),
    scratch_types=[
        pltpu.VMEM_SHARED((_TILES, _NPAD), jnp.float32),
        pltpu.VMEM((_NPAD,), jnp.float32),
        pltpu.VMEM((_NCH, _CH), jnp.int32),
        pltpu.VMEM((_NZ,), jnp.float32),
        pltpu.VMEM((_NZ,), jnp.float32),
        pltpu.VMEM((_NZ, 16), jnp.float32),
    ],
)
def _deg_kernel(idx_hbm, zeros_hbm, out_hbm,
                spbuf, hist, idx, accv, tmpv, outrows):
    c = lax.axis_index("c")
    s = lax.axis_index("s")
    pltpu.sync_copy(zeros_hbm, hist)
    pltpu.sync_copy(idx_hbm.at[pl.ds((c * _TILES + s) * _NCH, _NCH)], idx)
    ones = jnp.ones((16,), jnp.float32)
    zeros = jnp.zeros((16,), jnp.float32)

    @pl.loop(0, _NCH)
    def _(j):
        for k in range(_CH // 16):
            iv = idx[j, pl.ds(k * 16, 16)]
            plsc.addupdate_scatter(hist, [iv], ones)

    pltpu.sync_copy(hist, spbuf.at[s])
    plsc.subcore_barrier()

    for m in range(_NZ // 16):
        accv[pl.ds(m * 16, 16)] = zeros
    for t in range(_TILES):
        pltpu.sync_copy(spbuf.at[t, pl.ds(s * _NZ, _NZ)], tmpv)
        for m in range(_NZ // 16):
            sl = pl.ds(m * 16, 16)
            accv[sl] = accv[sl] + tmpv[sl]

    iota = lax.iota(jnp.int32, 16)
    for m in range(_NZ // 16):
        v = accv[pl.ds(m * 16, 16)]
        rows = jnp.full((16,), m * 16, jnp.int32) + iota
        for r in range(16):
            plsc.store_scatter(outrows, [rows, jnp.full((16,), r, jnp.int32)], v)

    pltpu.sync_copy(outrows,
                    out_hbm.at[pl.ds(c * _NPAD + s * _NZ, _NZ)])


_ACC = 10112
_NZA = _ACC // _TILES
_PC = 40


def _make_agg(phases, split_sidx):
    @functools.partial(
        pl.kernel,
        out_type=jax.ShapeDtypeStruct((_NCORES * _NPAD, 128), jnp.float32),
        mesh=_sc_mesh,
        scratch_types=[
            pltpu.VMEM_SHARED((_ACC, 128), jnp.float32),
            pltpu.VMEM((_PC, _CH), jnp.int32),
            pltpu.VMEM((_PC, _CH), jnp.int32),
            pltpu.VMEM((_CH, 128), jnp.float32),
            pltpu.VMEM((_CH, 128), jnp.float32),
            pltpu.SemaphoreType.DMA,
            pltpu.SemaphoreType.DMA,
            pltpu.SemaphoreType.DMA,
            pltpu.SemaphoreType.DMA,
        ],
    )
    def agg(p_hbm, gidx_hbm, sidx_hbm, zeros_hbm, out_hbm,
            acc, gidx, sidx, rows0, rows1, gs0, gs1, ss0, ss1):
        c = lax.axis_index("c")
        s = lax.axis_index("s")
        w = c * _TILES + s
        pltpu.sync_copy(zeros_hbm, acc.at[pl.ds(s * _NZA, _NZA)])
        plsc.subcore_barrier()

        def wait_g(buf, gsem):
            pltpu.make_async_copy(p_hbm.at[gidx.at[0]], buf, gsem).wait()

        def wait_s(buf, ssem):
            pltpu.make_async_copy(buf, acc.at[sidx.at[0]], ssem).wait()

        for p in range(phases):
            pltpu.sync_copy(gidx_hbm.at[pl.ds((w * phases + p) * _PC, _PC)],
                            gidx)
            sb = ((w if split_sidx else s) * phases + p) * _PC
            pltpu.sync_copy(sidx_hbm.at[pl.ds(sb, _PC)], sidx)

            pltpu.async_copy(p_hbm.at[gidx.at[0]], rows0, gs0)
            wait_g(rows0, gs0)
            pltpu.async_copy(rows0, acc.at[sidx.at[0]], ss0, add=True)
            pltpu.async_copy(p_hbm.at[gidx.at[1]], rows1, gs1)
            wait_g(rows1, gs1)
            pltpu.async_copy(rows1, acc.at[sidx.at[1]], ss1, add=True)
            wait_s(rows0, ss0)
            pltpu.async_copy(p_hbm.at[gidx.at[2]], rows0, gs0)

            @pl.loop(1, _PC // 2 - 1)
            def _(jj):
                j0 = jj * 2
                wait_g(rows0, gs0)
                pltpu.async_copy(rows0, acc.at[sidx.at[j0]], ss0, add=True)
                wait_s(rows1, ss1)
                pltpu.async_copy(p_hbm.at[gidx.at[j0 + 1]], rows1, gs1)
                wait_g(rows1, gs1)
                pltpu.async_copy(rows1, acc.at[sidx.at[j0 + 1]], ss1, add=True)
                wait_s(rows0, ss0)
                pltpu.async_copy(p_hbm.at[gidx.at[j0 + 2]], rows0, gs0)

            wait_g(rows0, gs0)
            pltpu.async_copy(rows0, acc.at[sidx.at[_PC - 2]], ss0, add=True)
            wait_s(rows1, ss1)
            pltpu.async_copy(p_hbm.at[gidx.at[_PC - 1]], rows1, gs1)
            wait_g(rows1, gs1)
            pltpu.async_copy(rows1, acc.at[sidx.at[_PC - 1]], ss1, add=True)
            wait_s(rows0, ss0)
            wait_s(rows1, ss1)

        plsc.subcore_barrier()
        pltpu.sync_copy(acc.at[pl.ds(s * _NZA, _NZA)],
                        out_hbm.at[pl.ds(c * _NPAD + s * _NZA, _NZA)])

    return agg


_NCH2 = _NCH // 2
_agg128 = _make_agg(phases=2, split_sidx=False)
_agg2 = _make_agg(phases=1, split_sidx=True)


def _norm_col(deg16):
    n = jnp.where(deg16 > 0.0, lax.rsqrt(jnp.maximum(deg16, 1.0)), 0.0)
    return n[:, :1]


def _tc1_body(x_ref, degs_ref, w1_ref, w2_ref, wc_ref, p1_ref, wc2_ref):
    ns = _norm_col(degs_ref[0])
    p1 = jnp.dot(x_ref[...] * ns, w1_ref[...],
                 preferred_element_type=jnp.float32)
    p1_ref[0] = p1[:, :128]
    p1_ref[1] = p1[:, 128:]
    wc2_ref[...] = jnp.dot(w2_ref[...], wc_ref[...],
                           preferred_element_type=jnp.float32)


_tc1 = pl.pallas_call(
    _tc1_body,
    grid=(_NPAD // _BLK,),
    in_specs=[
        pl.BlockSpec((_BLK, _D), lambda i: (i, 0)),
        pl.BlockSpec((1, _BLK, 16), lambda i: (0, i, 0)),
        pl.BlockSpec((_D, _D), lambda i: (0, 0)),
        pl.BlockSpec((_D, _D), lambda i: (0, 0)),
        pl.BlockSpec((_D, _OUT), lambda i: (0, 0)),
    ],
    out_specs=[
        pl.BlockSpec((2, _BLK, 128), lambda i: (0, i, 0)),
        pl.BlockSpec((_D, _OUT), lambda i: (0, 0)),
    ],
    out_shape=[
        jax.ShapeDtypeStruct((2, _NPAD, 128), jnp.float32),
        jax.ShapeDtypeStruct((_D, _OUT), jnp.float32),
    ],
)


def _tc2_body(a_ref, degs_ref, degd_ref, b1_ref, g1_ref, be1_ref, wc2_ref,
              p2_ref):
    ns = _norm_col(degs_ref[0])
    nd = _norm_col(degd_ref[0])
    h0 = a_ref[0] * nd + b1_ref[0:1, :128]
    h1 = a_ref[1] * nd + b1_ref[0:1, 128:]
    mu = (jnp.sum(h0, -1, keepdims=True) +
          jnp.sum(h1, -1, keepdims=True)) * (1.0 / _D)
    c0 = h0 - mu
    c1 = h1 - mu
    var = (jnp.sum(c0 * c0, -1, keepdims=True) +
           jnp.sum(c1 * c1, -1, keepdims=True)) * (1.0 / _D)
    inv = lax.rsqrt(var + 1e-5)
    y0 = jnp.maximum(c0 * inv * g1_ref[0:1, :128] + be1_ref[0:1, :128],
                     0.0) * ns
    y1 = jnp.maximum(c1 * inv * g1_ref[0:1, 128:] + be1_ref[0:1, 128:],
                     0.0) * ns
    p2 = (jnp.dot(y0, wc2_ref[:128], preferred_element_type=jnp.float32) +
          jnp.dot(y1, wc2_ref[128:], preferred_element_type=jnp.float32))
    p2_ref[...] = jnp.concatenate(
        [p2, jnp.zeros((p2.shape[0], 128 - _OUT), jnp.float32)], axis=-1)


_tc2 = pl.pallas_call(
    _tc2_body,
    grid=(_NPAD // _BLK,),
    in_specs=[
        pl.BlockSpec((2, _BLK, 128), lambda i: (0, i, 0)),
        pl.BlockSpec((1, _BLK, 16), lambda i: (0, i, 0)),
        pl.BlockSpec((1, _BLK, 16), lambda i: (1, i, 0)),
        pl.BlockSpec((1, _D), lambda i: (0, 0)),
        pl.BlockSpec((1, _D), lambda i: (0, 0)),
        pl.BlockSpec((1, _D), lambda i: (0, 0)),
        pl.BlockSpec((_D, _OUT), lambda i: (0, 0)),
    ],
    out_specs=pl.BlockSpec((_BLK, 128), lambda i: (i, 0)),
    out_shape=jax.ShapeDtypeStruct((_NPAD, 128), jnp.float32),
)


def _tc3_body(a_ref, degd_ref, b2_ref, wc_ref, bc_ref, out_ref):
    nd = _norm_col(degd_ref[0])
    a2 = (a_ref[0] + a_ref[1])[:, :_OUT] * nd
    bc2 = jnp.dot(b2_ref[...], wc_ref[...],
                  preferred_element_type=jnp.float32) + bc_ref[...]
    out_ref[...] = a2 + bc2


_tc3 = pl.pallas_call(
    _tc3_body,
    grid=(_NPAD // _BLK,),
    in_specs=[
        pl.BlockSpec((2, _BLK, 128), lambda i: (0, i, 0)),
        pl.BlockSpec((1, _BLK, 16), lambda i: (1, i, 0)),
        pl.BlockSpec((1, _D), lambda i: (0, 0)),
        pl.BlockSpec((_D, _OUT), lambda i: (0, 0)),
        pl.BlockSpec((1, _OUT), lambda i: (0, 0)),
    ],
    out_specs=pl.BlockSpec((_BLK, _OUT), lambda i: (i, 0)),
    out_shape=jax.ShapeDtypeStruct((_NPAD, _OUT), jnp.float32),
)


def kernel(x, edge_index, W1, b1, g1, be1, W2, b2, Wc, bc):
    src = edge_index[0]
    dst = edge_index[1]
    pad = _N + (jnp.arange(_EPAD - _E, dtype=jnp.int32) % (_ACC - _N))
    srcp = jnp.concatenate([src, pad])
    dstp = jnp.concatenate([dst, pad])

    deg_idx = jnp.concatenate([srcp, dstp]).reshape(2 * _ROWS, _CH)
    gidx1 = jnp.concatenate([srcp, srcp + _NPAD]).reshape(2 * _ROWS, _CH)
    src_rows = srcp.reshape(_ROWS, _CH)
    dst_rows = dstp.reshape(_ROWS, _CH)

    zeros1d = jnp.zeros((_NPAD,), jnp.float32)
    zeros128 = jnp.zeros((_NZA, 128), jnp.float32)

    degs = _deg_kernel(deg_idx, zeros1d).reshape(_NCORES, _NPAD, 16)

    xp = jnp.pad(x, ((0, _NPAD - _N), (0, 0)))
    p1, wc2 = _tc1(xp, degs, W1, W2, Wc)

    a1 = _agg128(p1.reshape(2 * _NPAD, 128), gidx1, dst_rows,
                 zeros128).reshape(2, _NPAD, 128)

    p2 = _tc2(a1, degs, degs, b1.reshape(1, _D), g1.reshape(1, _D),
              be1.reshape(1, _D), wc2)

    a2 = _agg2(p2, src_rows, dst_rows, zeros128).reshape(2, _NPAD, 128)

    logits = _tc3(a2, degs, b2.reshape(1, _D), Wc, bc.reshape(1, _OUT))
    return logits[:_N]
